# baseline (speedup 1.0000x reference)
import dataclasses

import jax
import jax.numpy as jnp
from jax import lax
from jax.experimental import pallas as pl
from jax.experimental.pallas import tpu as pltpu
from jax.experimental.pallas import tpu_sc as plsc

T = 4
N = 10000
D = 128
HID = 128
E = 320000

NC = 2
NS = 16
LANES = 16
FH = HID // NC

EPW = E // NS
CHUNK = 112
EPW_PAD = 20160
NCHUNK = EPW_PAD // CHUNK
NBUF = 2
NGRP = NCHUNK // NBUF
ROWS_PW = 624
ZROWS = 208
NTAIL = N - NS * ROWS_PW


def _sc_agg_body(h_hbm, src_hbm, dst_hbm, w_hbm, out_hbm,
                 src_v, dst_v, w_v, rows0, rows1, zbuf, acc,
                 gsem, ssem):
    cid = lax.axis_index("c")
    sid = lax.axis_index("s")
    rows = [rows0, rows1]

    zv = jnp.zeros((LANES,), jnp.float32)

    @pl.loop(0, ZROWS)
    def _(r):
        for k in range(FH // LANES):
            zbuf.at[r, pl.ds(k * LANES, LANES)][...] = zv

    base = sid * ROWS_PW

    @pl.loop(0, T)
    def _(t):
        pltpu.sync_copy(src_hbm.at[t, sid], src_v)
        pltpu.sync_copy(dst_hbm.at[t, sid], dst_v)
        pltpu.sync_copy(w_hbm.at[t, sid], w_v)
        for z in range(ROWS_PW // ZROWS):
            pltpu.sync_copy(zbuf, acc.at[pl.ds(base + z * ZROWS, ZROWS)])

        @pl.when(sid == NS - 1)
        def _():
            pltpu.sync_copy(zbuf.at[pl.ds(0, NTAIL)],
                            acc.at[pl.ds(NS * ROWS_PW, NTAIL)])

        plsc.subcore_barrier()

        def _scale(buf, ci):
            @plsc.parallel_loop(0, CHUNK // LANES, unroll=2)
            def _(g):
                w16 = w_v.at[ci, pl.ds(g * LANES, LANES)][...]
                for j in range(LANES):
                    wb = lax.gather(
                        w16, jnp.full((LANES, 1), j, jnp.int32),
                        dimension_numbers=lax.GatherDimensionNumbers(
                            offset_dims=(), collapsed_slice_dims=(0,),
                            start_index_map=(0,)),
                        slice_sizes=(1,),
                        mode=lax.GatherScatterMode.PROMISE_IN_BOUNDS)
                    r = g * LANES + j
                    for k in range(FH // LANES):
                        sl = pl.ds(k * LANES, LANES)
                        buf.at[r, sl][...] = buf.at[r, sl][...] * wb

        def _gather(buf, ci, b):
            pltpu.async_copy(h_hbm.at[cid].at[src_v.at[ci]], buf, gsem.at[b])

        for b in range(NBUF):
            _gather(rows[b], b, b)

        @pl.loop(0, NGRP)
        def _(j):
            for b in range(NBUF):
                ci = j * NBUF + b
                pltpu.make_async_copy(h_hbm.at[cid].at[src_v.at[ci]],
                                      rows[b], gsem.at[b]).wait()
                _scale(rows[b], ci)
                pltpu.async_copy(rows[b], acc.at[dst_v.at[ci]], ssem.at[b],
                                 add=True)

            @pl.when(j < NGRP - 1)
            def _():
                for b in range(NBUF):
                    ci = j * NBUF + b
                    pltpu.make_async_copy(rows[b], acc.at[dst_v.at[ci]],
                                          ssem.at[b]).wait()
                    _gather(rows[b], ci + NBUF, b)

        for b in range(NBUF):
            ci = NCHUNK - NBUF + b
            pltpu.make_async_copy(rows[b], acc.at[dst_v.at[ci]],
                                  ssem.at[b]).wait()

        plsc.subcore_barrier()
        for z in range(ROWS_PW // ZROWS):
            sl = pl.ds(base + z * ZROWS, ZROWS)
            pltpu.sync_copy(acc.at[sl], out_hbm.at[cid, t, sl])

        @pl.when(sid == NS - 1)
        def _():
            sl = pl.ds(NS * ROWS_PW, NTAIL)
            pltpu.sync_copy(acc.at[sl], out_hbm.at[cid, t, sl])


def _sc_compiler_params():
    cp = pltpu.CompilerParams()
    if "needs_layout_passes" in pltpu.CompilerParams.__dataclass_fields__:
        cp = dataclasses.replace(cp, needs_layout_passes=False)
    cp = dataclasses.replace(cp, use_tc_tiling_on_sc=False)
    return cp


_SC_AGG_KERNEL = None


def _sc_aggregate(h_halves, src_g, dst, w):
    global _SC_AGG_KERNEL
    if _SC_AGG_KERNEL is None:
        _SC_AGG_KERNEL = pl.kernel(
            _sc_agg_body,
            out_type=jax.ShapeDtypeStruct((NC, T, N, FH), jnp.float32),
            mesh=plsc.VectorSubcoreMesh(core_axis_name="c",
                                        subcore_axis_name="s"),
            scratch_types=[
                pltpu.VMEM((NCHUNK, CHUNK), jnp.int32),
                pltpu.VMEM((NCHUNK, CHUNK), jnp.int32),
                pltpu.VMEM((NCHUNK, CHUNK), jnp.float32),
                pltpu.VMEM((CHUNK, FH), jnp.float32),
                pltpu.VMEM((CHUNK, FH), jnp.float32),
                pltpu.VMEM((ZROWS, FH), jnp.float32),
                pltpu.VMEM_SHARED((N, FH), jnp.float32),
                pltpu.SemaphoreType.DMA((NBUF,)),
                pltpu.SemaphoreType.DMA((NBUF,)),
            ],
            compiler_params=_sc_compiler_params(),
        )
    return _SC_AGG_KERNEL(h_halves, src_g, dst, w)


def _mm_body(x_ref, w_ref, o_ref):
    r = jnp.dot(x_ref[...], w_ref[...], preferred_element_type=jnp.float32)
    o_ref[0] = r[:, :FH]
    o_ref[1] = r[:, FH:]


def _matmul_split(x, w):
    m, k = x.shape
    bm = 800
    return pl.pallas_call(
        _mm_body,
        grid=(m // bm,),
        in_specs=[pl.BlockSpec((bm, k), lambda i: (i, 0)),
                  pl.BlockSpec((k, HID), lambda i: (0, 0))],
        out_specs=pl.BlockSpec((NC, bm, FH), lambda i: (0, i, 0)),
        out_shape=jax.ShapeDtypeStruct((NC, m, FH), jnp.float32),
    )(x, w)


def _lstm_steps(agg_ref, wi, wh, b):
    rows = agg_ref.shape[2]
    h = jnp.zeros((rows, HID), jnp.float32)
    c = jnp.zeros((rows, HID), jnp.float32)
    hs = []
    for t in range(T):
        s = jnp.maximum(
            jnp.concatenate([agg_ref[0, t], agg_ref[1, t]], axis=-1), 0.0)
        g = (jnp.dot(s, wi, preferred_element_type=jnp.float32)
             + jnp.dot(h, wh, preferred_element_type=jnp.float32) + b)
        i_g = jax.nn.sigmoid(g[:, 0:HID])
        f_g = jax.nn.sigmoid(g[:, HID:2 * HID])
        g_g = jnp.tanh(g[:, 2 * HID:3 * HID])
        o_g = jax.nn.sigmoid(g[:, 3 * HID:4 * HID])
        c = f_g * c + i_g * g_g
        h = o_g * jnp.tanh(c)
        hs.append(h)
    return hs


def _lstm_proj_body(agg_ref, wi_ref, wh_ref, b_ref, wn_ref, o_ref):
    hs = _lstm_steps(agg_ref, wi_ref[...], wh_ref[...], b_ref[...])
    wn = wn_ref[...]
    for t in range(T):
        r = jnp.dot(hs[t], wn, preferred_element_type=jnp.float32)
        o_ref[0, t] = r[:, :FH]
        o_ref[1, t] = r[:, FH:]


def _lstm_final_body(agg_ref, wi_ref, wh_ref, b_ref, o_ref):
    hs = _lstm_steps(agg_ref, wi_ref[...], wh_ref[...], b_ref[...])
    for t in range(T):
        o_ref[:, t, :] = hs[t]


_BR = 1000


def _lstm_project(agg, wi, wh, b, wn):
    return pl.pallas_call(
        _lstm_proj_body,
        grid=(N // _BR,),
        in_specs=[pl.BlockSpec((NC, T, _BR, FH), lambda i: (0, 0, i, 0)),
                  pl.BlockSpec(wi.shape, lambda i: (0, 0)),
                  pl.BlockSpec(wh.shape, lambda i: (0, 0)),
                  pl.BlockSpec((1, 4 * HID), lambda i: (0, 0)),
                  pl.BlockSpec(wn.shape, lambda i: (0, 0))],
        out_specs=pl.BlockSpec((NC, T, _BR, FH), lambda i: (0, 0, i, 0)),
        out_shape=jax.ShapeDtypeStruct((NC, T, N, FH), jnp.float32),
    )(agg, wi, wh, b.reshape(1, -1), wn)


def _lstm_final(agg, wi, wh, b):
    return pl.pallas_call(
        _lstm_final_body,
        grid=(N // _BR,),
        in_specs=[pl.BlockSpec((NC, T, _BR, FH), lambda i: (0, 0, i, 0)),
                  pl.BlockSpec(wi.shape, lambda i: (0, 0)),
                  pl.BlockSpec(wh.shape, lambda i: (0, 0)),
                  pl.BlockSpec((1, 4 * HID), lambda i: (0, 0))],
        out_specs=pl.BlockSpec((_BR, T, HID), lambda i: (i, 0, 0)),
        out_shape=jax.ShapeDtypeStruct((N, T, HID), jnp.float32),
    )(agg, wi, wh, b.reshape(1, -1))


def kernel(x, edge_index, edge_weight, W0, W1, Wi0, Wh0, b0, Wi1, Wh1, b1):
    ei = edge_index.astype(jnp.int32)
    src = ei[:, 1, :]
    dst = ei[:, 0, :]
    offs = (jnp.arange(T, dtype=jnp.int32) * N)[:, None]
    pad = ((0, 0), (0, 0), (0, EPW_PAD - EPW))

    def _shard(a):
        return jnp.pad(a.reshape(T, NS, EPW), pad).reshape(
            T, NS, NCHUNK, CHUNK)

    src_g = _shard(src + offs)
    dst_r = _shard(dst)
    w_r = _shard(edge_weight.astype(jnp.float32))

    h0 = _matmul_split(x.reshape(T * N, D), W0)
    agg0 = _sc_aggregate(h0, src_g, dst_r, w_r)
    h1 = _lstm_project(agg0, Wi0, Wh0, b0, W1)
    agg1 = _sc_aggregate(h1.reshape(NC, T * N, FH), src_g, dst_r, w_r)
    return _lstm_final(agg1, Wi1, Wh1, b1)


# speedup vs baseline: 1.1055x; 1.1055x over previous
import dataclasses

import jax
import jax.numpy as jnp
from jax import lax
from jax.experimental import pallas as pl
from jax.experimental.pallas import tpu as pltpu
from jax.experimental.pallas import tpu_sc as plsc

T = 4
N = 10000
D = 128
HID = 128
E = 320000

NC = 2
NS = 16
LANES = 16
FH = HID // NC

EPW = E // NS
CHUNK = 112
EPW_PAD = 20160
NCHUNK = EPW_PAD // CHUNK
NBUF = 3
NGRP = NCHUNK // NBUF
ROWS_PW = 624
ZROWS = 104
NTAIL = N - NS * ROWS_PW


def _sc_agg_body(h_hbm, src_hbm, dst_hbm, w_hbm, out_hbm,
                 src_v, dst_v, w_v, rows0, rows1, rows2, zbuf, acc,
                 gsem, ssem):
    cid = lax.axis_index("c")
    sid = lax.axis_index("s")
    rows = [rows0, rows1, rows2]

    zv = jnp.zeros((LANES,), jnp.float32)

    @pl.loop(0, ZROWS)
    def _(r):
        for k in range(FH // LANES):
            zbuf.at[r, pl.ds(k * LANES, LANES)][...] = zv

    base = sid * ROWS_PW

    @pl.loop(0, T)
    def _(t):
        pltpu.sync_copy(src_hbm.at[t, sid], src_v)
        pltpu.sync_copy(dst_hbm.at[t, sid], dst_v)
        pltpu.sync_copy(w_hbm.at[t, sid], w_v)
        for z in range(ROWS_PW // ZROWS):
            pltpu.sync_copy(zbuf, acc.at[pl.ds(base + z * ZROWS, ZROWS)])

        @pl.when(sid == NS - 1)
        def _():
            pltpu.sync_copy(zbuf.at[pl.ds(0, NTAIL)],
                            acc.at[pl.ds(NS * ROWS_PW, NTAIL)])

        plsc.subcore_barrier()

        def _scale(buf, ci):
            @plsc.parallel_loop(0, CHUNK // LANES, unroll=2)
            def _(g):
                w16 = w_v.at[ci, pl.ds(g * LANES, LANES)][...]
                for j in range(LANES):
                    wb = lax.gather(
                        w16, jnp.full((LANES, 1), j, jnp.int32),
                        dimension_numbers=lax.GatherDimensionNumbers(
                            offset_dims=(), collapsed_slice_dims=(0,),
                            start_index_map=(0,)),
                        slice_sizes=(1,),
                        mode=lax.GatherScatterMode.PROMISE_IN_BOUNDS)
                    r = g * LANES + j
                    for k in range(FH // LANES):
                        sl = pl.ds(k * LANES, LANES)
                        buf.at[r, sl][...] = buf.at[r, sl][...] * wb

        def _gather(buf, ci, b):
            pltpu.async_copy(h_hbm.at[cid].at[src_v.at[ci]], buf, gsem.at[b])

        for b in range(NBUF):
            _gather(rows[b], b, b)

        @pl.loop(0, NGRP)
        def _(j):
            for b in range(NBUF):
                ci = j * NBUF + b
                pltpu.make_async_copy(h_hbm.at[cid].at[src_v.at[ci]],
                                      rows[b], gsem.at[b]).wait()
                _scale(rows[b], ci)
                pltpu.async_copy(rows[b], acc.at[dst_v.at[ci]], ssem.at[b],
                                 add=True)

            @pl.when(j < NGRP - 1)
            def _():
                for b in range(NBUF):
                    ci = j * NBUF + b
                    pltpu.make_async_copy(rows[b], acc.at[dst_v.at[ci]],
                                          ssem.at[b]).wait()
                    _gather(rows[b], ci + NBUF, b)

        for b in range(NBUF):
            ci = NCHUNK - NBUF + b
            pltpu.make_async_copy(rows[b], acc.at[dst_v.at[ci]],
                                  ssem.at[b]).wait()

        plsc.subcore_barrier()
        for z in range(ROWS_PW // ZROWS):
            sl = pl.ds(base + z * ZROWS, ZROWS)
            pltpu.sync_copy(acc.at[sl], out_hbm.at[cid, t, sl])

        @pl.when(sid == NS - 1)
        def _():
            sl = pl.ds(NS * ROWS_PW, NTAIL)
            pltpu.sync_copy(acc.at[sl], out_hbm.at[cid, t, sl])


def _sc_compiler_params():
    cp = pltpu.CompilerParams()
    if "needs_layout_passes" in pltpu.CompilerParams.__dataclass_fields__:
        cp = dataclasses.replace(cp, needs_layout_passes=False)
    cp = dataclasses.replace(cp, use_tc_tiling_on_sc=False)
    return cp


_SC_AGG_KERNEL = None


def _sc_aggregate(h_halves, src_g, dst, w):
    global _SC_AGG_KERNEL
    if _SC_AGG_KERNEL is None:
        _SC_AGG_KERNEL = pl.kernel(
            _sc_agg_body,
            out_type=jax.ShapeDtypeStruct((NC, T, N, FH), jnp.float32),
            mesh=plsc.VectorSubcoreMesh(core_axis_name="c",
                                        subcore_axis_name="s"),
            scratch_types=[
                pltpu.VMEM((NCHUNK, CHUNK), jnp.int32),
                pltpu.VMEM((NCHUNK, CHUNK), jnp.int32),
                pltpu.VMEM((NCHUNK, CHUNK), jnp.float32),
                pltpu.VMEM((CHUNK, FH), jnp.float32),
                pltpu.VMEM((CHUNK, FH), jnp.float32),
                pltpu.VMEM((CHUNK, FH), jnp.float32),
                pltpu.VMEM((ZROWS, FH), jnp.float32),
                pltpu.VMEM_SHARED((N, FH), jnp.float32),
                pltpu.SemaphoreType.DMA((NBUF,)),
                pltpu.SemaphoreType.DMA((NBUF,)),
            ],
            compiler_params=_sc_compiler_params(),
        )
    return _SC_AGG_KERNEL(h_halves, src_g, dst, w)


def _mm_body(x_ref, w_ref, o_ref):
    r = jnp.dot(x_ref[...], w_ref[...], preferred_element_type=jnp.float32)
    o_ref[0] = r[:, :FH]
    o_ref[1] = r[:, FH:]


def _matmul_split(x, w):
    m, k = x.shape
    bm = 800
    return pl.pallas_call(
        _mm_body,
        grid=(m // bm,),
        in_specs=[pl.BlockSpec((bm, k), lambda i: (i, 0)),
                  pl.BlockSpec((k, HID), lambda i: (0, 0))],
        out_specs=pl.BlockSpec((NC, bm, FH), lambda i: (0, i, 0)),
        out_shape=jax.ShapeDtypeStruct((NC, m, FH), jnp.float32),
    )(x, w)


def _lstm_steps(agg_ref, wi, wh, b):
    rows = agg_ref.shape[2]
    h = jnp.zeros((rows, HID), jnp.float32)
    c = jnp.zeros((rows, HID), jnp.float32)
    hs = []
    for t in range(T):
        s = jnp.maximum(
            jnp.concatenate([agg_ref[0, t], agg_ref[1, t]], axis=-1), 0.0)
        g = (jnp.dot(s, wi, preferred_element_type=jnp.float32)
             + jnp.dot(h, wh, preferred_element_type=jnp.float32) + b)
        i_g = jax.nn.sigmoid(g[:, 0:HID])
        f_g = jax.nn.sigmoid(g[:, HID:2 * HID])
        g_g = jnp.tanh(g[:, 2 * HID:3 * HID])
        o_g = jax.nn.sigmoid(g[:, 3 * HID:4 * HID])
        c = f_g * c + i_g * g_g
        h = o_g * jnp.tanh(c)
        hs.append(h)
    return hs


def _lstm_proj_body(agg_ref, wi_ref, wh_ref, b_ref, wn_ref, o_ref):
    hs = _lstm_steps(agg_ref, wi_ref[...], wh_ref[...], b_ref[...])
    wn = wn_ref[...]
    for t in range(T):
        r = jnp.dot(hs[t], wn, preferred_element_type=jnp.float32)
        o_ref[0, t] = r[:, :FH]
        o_ref[1, t] = r[:, FH:]


def _lstm_final_body(agg_ref, wi_ref, wh_ref, b_ref, o_ref):
    hs = _lstm_steps(agg_ref, wi_ref[...], wh_ref[...], b_ref[...])
    for t in range(T):
        o_ref[:, t, :] = hs[t]


_BR = 1000


def _lstm_project(agg, wi, wh, b, wn):
    return pl.pallas_call(
        _lstm_proj_body,
        grid=(N // _BR,),
        in_specs=[pl.BlockSpec((NC, T, _BR, FH), lambda i: (0, 0, i, 0)),
                  pl.BlockSpec(wi.shape, lambda i: (0, 0)),
                  pl.BlockSpec(wh.shape, lambda i: (0, 0)),
                  pl.BlockSpec((1, 4 * HID), lambda i: (0, 0)),
                  pl.BlockSpec(wn.shape, lambda i: (0, 0))],
        out_specs=pl.BlockSpec((NC, T, _BR, FH), lambda i: (0, 0, i, 0)),
        out_shape=jax.ShapeDtypeStruct((NC, T, N, FH), jnp.float32),
    )(agg, wi, wh, b.reshape(1, -1), wn)


def _lstm_final(agg, wi, wh, b):
    return pl.pallas_call(
        _lstm_final_body,
        grid=(N // _BR,),
        in_specs=[pl.BlockSpec((NC, T, _BR, FH), lambda i: (0, 0, i, 0)),
                  pl.BlockSpec(wi.shape, lambda i: (0, 0)),
                  pl.BlockSpec(wh.shape, lambda i: (0, 0)),
                  pl.BlockSpec((1, 4 * HID), lambda i: (0, 0))],
        out_specs=pl.BlockSpec((_BR, T, HID), lambda i: (i, 0, 0)),
        out_shape=jax.ShapeDtypeStruct((N, T, HID), jnp.float32),
    )(agg, wi, wh, b.reshape(1, -1))


def kernel(x, edge_index, edge_weight, W0, W1, Wi0, Wh0, b0, Wi1, Wh1, b1):
    ei = edge_index.astype(jnp.int32)
    src = ei[:, 1, :]
    dst = ei[:, 0, :]
    offs = (jnp.arange(T, dtype=jnp.int32) * N)[:, None]
    pad = ((0, 0), (0, 0), (0, EPW_PAD - EPW))

    def _shard(a):
        return jnp.pad(a.reshape(T, NS, EPW), pad).reshape(
            T, NS, NCHUNK, CHUNK)

    src_g = _shard(src + offs)
    dst_r = _shard(dst)
    w_r = _shard(edge_weight.astype(jnp.float32))

    h0 = _matmul_split(x.reshape(T * N, D), W0)
    agg0 = _sc_aggregate(h0, src_g, dst_r, w_r)
    h1 = _lstm_project(agg0, Wi0, Wh0, b0, W1)
    agg1 = _sc_aggregate(h1.reshape(NC, T * N, FH), src_g, dst_r, w_r)
    return _lstm_final(agg1, Wi1, Wh1, b1)
